# Initial kernel scaffold; baseline (speedup 1.0000x reference)
import jax
import jax.numpy as jnp
from jax.experimental import pallas as pl


def kernel(preds, targets):
    raise NotImplementedError("write your pallas kernel here")


# speedup vs baseline: 10.3308x; 10.3308x over previous
import functools

import jax
import jax.numpy as jnp
from jax import lax
from jax.experimental import pallas as pl
from jax.experimental.pallas import tpu as pltpu
from jax.experimental.pallas import tpu_sc as plsc

B = 4
N = 20000
M = 32
LANES = 16
NW = 32
PAIRS_PER_W = 4
CHUNKS = N // LANES
IOU_THR = 0.2
BIG_I32 = 2 ** 30

@functools.cache
def _get_sc_match():
    mesh = plsc.VectorSubcoreMesh(core_axis_name="c", subcore_axis_name="s")

    @functools.partial(
        pl.kernel,
        mesh=mesh,
        out_type=(
            jax.ShapeDtypeStruct((NW, LANES), jnp.float32),
            jax.ShapeDtypeStruct((NW, LANES), jnp.int32),
            jax.ShapeDtypeStruct((NW, PAIRS_PER_W, LANES), jnp.float32),
        ),
        scratch_types=(
            pltpu.VMEM((N * 5,), jnp.float32),
            pltpu.VMEM((LANES,), jnp.float32),
            pltpu.VMEM((LANES,), jnp.float32),
            pltpu.VMEM((LANES,), jnp.int32),
            pltpu.VMEM((PAIRS_PER_W, LANES), jnp.float32),
        ),
        compiler_params=pltpu.CompilerParams(needs_layout_passes=False),
    )
    def _sc_match(preds_hbm, targets_hbm, val_out, idx_out, rows_out,
                  pred_v, tgt_v, valrow_v, idxrow_v, rows_v):
        wid = lax.axis_index("s") * 2 + lax.axis_index("c")
        b = wid // 8
        j0 = (wid % 8) * PAIRS_PER_W

        pltpu.sync_copy(preds_hbm.at[b], pred_v)
        pltpu.sync_copy(targets_hbm.at[b, pl.ds(j0 * 4, LANES)], tgt_v)

        iota = lax.iota(jnp.int32, LANES)
        iota5 = iota * 5

        tvec = tgt_v[...]
        gx1 = [tvec[g * 4 + 0] for g in range(PAIRS_PER_W)]
        gy1 = [tvec[g * 4 + 1] for g in range(PAIRS_PER_W)]
        gx2 = [tvec[g * 4 + 2] for g in range(PAIRS_PER_W)]
        gy2 = [tvec[g * 4 + 3] for g in range(PAIRS_PER_W)]
        garea = [(gx2[g] - gx1[g]) * (gy2[g] - gy1[g]) for g in range(PAIRS_PER_W)]

        def chunk_body(c, carry):
            bvs, bis = carry
            rows_i = c * LANES + iota
            base = c * (LANES * 5) + iota5
            cx = plsc.load_gather(pred_v, [base])
            cy = plsc.load_gather(pred_v, [base + 1])
            w = jnp.maximum(plsc.load_gather(pred_v, [base + 2]), 0.0001)
            h = jnp.maximum(plsc.load_gather(pred_v, [base + 3]), 0.0001)
            x1 = cx - w / 2
            y1 = cy - h / 2
            x2 = cx + w / 2
            y2 = cy + h / 2
            area1 = (x2 - x1) * (y2 - y1)
            new_bvs = []
            new_bis = []
            for g in range(PAIRS_PER_W):
                iw = jnp.maximum(jnp.minimum(x2, gx2[g]) - jnp.maximum(x1, gx1[g]), 0.0)
                ih = jnp.maximum(jnp.minimum(y2, gy2[g]) - jnp.maximum(y1, gy1[g]), 0.0)
                inter = iw * ih
                iou = inter / (area1 + garea[g] - inter)
                upd = iou > bvs[g]
                new_bvs.append(jnp.where(upd, iou, bvs[g]))
                new_bis.append(jnp.where(upd, rows_i, bis[g]))
            return tuple(new_bvs), tuple(new_bis)

        init = (
            tuple(jnp.full((LANES,), -jnp.inf, jnp.float32) for _ in range(PAIRS_PER_W)),
            tuple(jnp.zeros((LANES,), jnp.int32) for _ in range(PAIRS_PER_W)),
        )
        bvs, bis = lax.fori_loop(0, CHUNKS, chunk_body, init)

        valrow = jnp.zeros((LANES,), jnp.float32)
        idxrow = jnp.zeros((LANES,), jnp.int32)
        gcol = jnp.minimum(iota, 4)
        for g in range(PAIRS_PER_W):
            m = jnp.max(bvs[g])
            mi = jnp.min(jnp.where(bvs[g] == m, bis[g], BIG_I32))
            valrow = jnp.where(iota == g, m, valrow)
            idxrow = jnp.where(iota == g, mi, idxrow)
            rows_v[g, :] = plsc.load_gather(pred_v, [mi * 5 + gcol])

        valrow_v[...] = valrow
        idxrow_v[...] = idxrow
        pltpu.sync_copy(valrow_v, val_out.at[wid])
        pltpu.sync_copy(idxrow_v, idx_out.at[wid])
        pltpu.sync_copy(rows_v, rows_out.at[wid])

    return _sc_match


_ATAN_COEFFS = (
    0.9999999828647296, -0.3333319654947794, 0.19996761628871834,
    -0.1425013453688434, 0.10891953602946322, -0.08252553527101676,
    0.055674573852792425, -0.029126338688523788, 0.009906944501302792,
    -0.0015853086116741147,
)


def _atan(x):
    ax = jnp.abs(x)
    inv = ax > 1.0
    z = jnp.where(inv, 1.0 / ax, ax)
    z2 = z * z
    acc = jnp.full_like(z, _ATAN_COEFFS[-1])
    for c in _ATAN_COEFFS[-2::-1]:
        acc = acc * z2 + c
    p = acc * z
    r = jnp.where(inv, jnp.float32(jnp.pi / 2) - p, p)
    return jnp.where(x < 0, -r, r)


def _focal_elem(x, t):
    prob = 1.0 / (1.0 + jnp.exp(-x))
    ce = jnp.maximum(x, 0.0) - x * t + jnp.log1p(jnp.exp(-jnp.abs(x)))
    p_t = prob * t + (1.0 - prob) * (1.0 - t)
    alpha_t = 0.25 * t + 0.75 * (1.0 - t)
    p_t = jnp.clip(p_t, 1e-06, 1.0 - 1e-06)
    om = 1.0 - p_t
    return alpha_t * om * om * ce


def _tc_body(conf_ref, val_ref, idx_ref, mcx_ref, mcy_ref, mw_ref, mh_ref,
             mconf_ref, tx1_ref, ty1_ref, tx2_ref, ty2_ref, o_ref):
    conf = conf_ref[...]
    s0 = jnp.sum(_focal_elem(conf, 0.0), axis=1, keepdims=True)

    val = val_ref[...]
    idx = idx_ref[...]
    qual = val > IOU_THR
    eq = idx[:, :, None] == idx[:, None, :]
    jj = lax.broadcasted_iota(jnp.int32, (B, M, M), 1)
    jp = lax.broadcasted_iota(jnp.int32, (B, M, M), 2)
    blocked = jnp.any(eq & (jp < jj) & qual[:, None, :], axis=2)
    maskf = (qual & ~blocked).astype(jnp.float32)

    mc = mconf_ref[...]
    corr = jnp.sum(maskf * (_focal_elem(mc, 1.0) - _focal_elem(mc, 0.0)),
                   axis=1, keepdims=True)
    conf_loss = (s0 + corr) / jnp.float32(N)

    w = jnp.maximum(mw_ref[...], 0.0001)
    h = jnp.maximum(mh_ref[...], 0.0001)
    x1 = mcx_ref[...] - w / 2
    y1 = mcy_ref[...] - h / 2
    x2 = mcx_ref[...] + w / 2
    y2 = mcy_ref[...] + h / 2
    x1g = tx1_ref[...]
    y1g = ty1_ref[...]
    x2g = tx2_ref[...]
    y2g = ty2_ref[...]
    eps = 1e-07
    intsct = (jnp.maximum(jnp.minimum(x2, x2g) - jnp.maximum(x1, x1g), 0.0)
              * jnp.maximum(jnp.minimum(y2, y2g) - jnp.maximum(y1, y1g), 0.0))
    union = (x2 - x1) * (y2 - y1) + (x2g - x1g) * (y2g - y1g) - intsct + eps
    iou = intsct / union
    diag = ((jnp.maximum(x2, x2g) - jnp.minimum(x1, x1g)) ** 2
            + (jnp.maximum(y2, y2g) - jnp.minimum(y1, y1g)) ** 2 + eps)
    dist = (((x1g + x2g) - (x1 + x2)) / 2) ** 2 + (((y1g + y2g) - (y1 + y2)) / 2) ** 2
    diou = 1.0 - iou + dist / diag
    v = (4.0 / (jnp.pi ** 2)) * (_atan((x2g - x1g) / (y2g - y1g))
                                 - _atan((x2 - x1) / (y2 - y1))) ** 2
    alpha = v / (1.0 - iou + v + eps)
    ciou = diou + alpha * v

    n_match = jnp.sum(maskf, axis=1, keepdims=True)
    box_loss = jnp.sum(ciou * maskf, axis=1, keepdims=True) / jnp.maximum(n_match, 1.0)
    per_b = conf_loss + jnp.where(n_match > 0.0, box_loss, 0.0)
    o_ref[...] = jnp.reshape(jnp.sum(per_b) / jnp.float32(B), (1, 1))


_tc_loss = pl.pallas_call(
    _tc_body,
    out_shape=jax.ShapeDtypeStruct((1, 1), jnp.float32),
)


def kernel(preds, targets):
    val_rows, idx_rows, row_rows = _get_sc_match()(
        preds.reshape(B, N * 5), targets.reshape(B, M * 4))
    val = val_rows[:, :PAIRS_PER_W].reshape(B, M)
    idx = idx_rows[:, :PAIRS_PER_W].reshape(B, M)
    rows = row_rows.reshape(B, M, LANES)
    mcx = rows[:, :, 0]
    mcy = rows[:, :, 1]
    mw = rows[:, :, 2]
    mh = rows[:, :, 3]
    mconf = rows[:, :, 4]
    conf = preds[:, :, 4]
    tx1 = targets[:, :, 0]
    ty1 = targets[:, :, 1]
    tx2 = targets[:, :, 2]
    ty2 = targets[:, :, 3]
    out = _tc_loss(conf, val, idx, mcx, mcy, mw, mh, mconf, tx1, ty1, tx2, ty2)
    return out[0, 0]


# speedup vs baseline: 24.4014x; 2.3620x over previous
import functools

import jax
import jax.numpy as jnp
from jax import lax
from jax.experimental import pallas as pl
from jax.experimental.pallas import tpu as pltpu
from jax.experimental.pallas import tpu_sc as plsc

B = 4
N = 20000
M = 32
LANES = 16
NW = 32
WPB = 8
PAIRS_PER_W = 4
CHUNKS = N // LANES
CONF_SLICE = N // WPB
CONF_CHUNKS = (CONF_SLICE + LANES - 1) // LANES
IOU_THR = 0.2
BIG_I32 = 2 ** 30

_ATAN_COEFFS = (
    0.9999999828647296, -0.3333319654947794, 0.19996761628871834,
    -0.1425013453688434, 0.10891953602946322, -0.08252553527101676,
    0.055674573852792425, -0.029126338688523788, 0.009906944501302792,
    -0.0015853086116741147,
)

_LOG1P_COEFFS = (
    0.9999999932119505, -0.49999957772713577, 0.33332362587735553,
    -0.24988381968477616, 0.19915963425828467, -0.16267261242156672,
    0.1297016934899417, -0.09379880994704069, 0.056020253063691836,
    -0.024756563748861746, 0.006977327185597779, -0.0009239630494390083,
)


def _atan(x):
    ax = jnp.abs(x)
    inv = ax > 1.0
    z = jnp.where(inv, 1.0 / ax, ax)
    z2 = z * z
    acc = jnp.full_like(z, _ATAN_COEFFS[-1])
    for c in _ATAN_COEFFS[-2::-1]:
        acc = acc * z2 + c
    p = acc * z
    r = jnp.where(inv, jnp.float32(jnp.pi / 2) - p, p)
    return jnp.where(x < 0, -r, r)


def _log1p01(z):
    acc = jnp.full_like(z, _LOG1P_COEFFS[-1])
    for c in _LOG1P_COEFFS[-2::-1]:
        acc = acc * z + c
    return acc * z


def _focal01(x):
    e = jnp.exp(-jnp.abs(x))
    ln1pe = _log1p01(e)
    ce0 = jnp.maximum(x, 0.0) + ln1pe
    prob = jnp.where(x >= 0, 1.0, e) / (1.0 + e)
    pt0 = jnp.clip(1.0 - prob, 1e-06, 1.0 - 1e-06)
    om0 = 1.0 - pt0
    l0 = 0.75 * om0 * om0 * ce0
    pt1 = jnp.clip(prob, 1e-06, 1.0 - 1e-06)
    om1 = 1.0 - pt1
    l1 = 0.25 * om1 * om1 * (ce0 - x)
    return l0, l1


@functools.cache
def _get_sc_loss():
    mesh = plsc.VectorSubcoreMesh(core_axis_name="c", subcore_axis_name="s")

    @functools.partial(
        pl.kernel,
        mesh=mesh,
        out_type=jax.ShapeDtypeStruct((NW, LANES), jnp.float32),
        scratch_types=(
            pltpu.VMEM((5 * N,), jnp.float32),
            pltpu.VMEM((LANES,), jnp.float32),
            pltpu.VMEM((LANES,), jnp.float32),
            pltpu.VMEM((LANES,), jnp.int32),
            pltpu.VMEM((WPB * LANES,), jnp.float32),
            pltpu.VMEM((WPB * LANES,), jnp.int32),
            pltpu.VMEM((LANES,), jnp.float32),
            pltpu.VMEM_SHARED((NW * LANES,), jnp.float32),
            pltpu.VMEM_SHARED((NW * LANES,), jnp.int32),
        ),
        compiler_params=pltpu.CompilerParams(needs_layout_passes=False),
    )
    def _sc_loss(preds_hbm, targets_hbm, part_out,
                 pred_v, tgt_v, valrow_v, idxrow_v, lval_v, lidx_v, part_v,
                 shval, shidx):
        wid = lax.axis_index("c") * 16 + lax.axis_index("s")
        b = wid // WPB
        jg = wid % WPB
        j0 = jg * PAIRS_PER_W

        for k in range(5):
            pltpu.sync_copy(preds_hbm.at[pl.ds((k * B + b) * N, N)],
                            pred_v.at[pl.ds(k * N, N)])
        pltpu.sync_copy(targets_hbm.at[b, pl.ds(j0 * 4, LANES)], tgt_v)

        iota = lax.iota(jnp.int32, LANES)
        lane_lt4 = iota < PAIRS_PER_W

        tvec = tgt_v[...]
        gx1 = [tvec[g * 4 + 0] for g in range(PAIRS_PER_W)]
        gy1 = [tvec[g * 4 + 1] for g in range(PAIRS_PER_W)]
        gx2 = [tvec[g * 4 + 2] for g in range(PAIRS_PER_W)]
        gy2 = [tvec[g * 4 + 3] for g in range(PAIRS_PER_W)]
        garea = [(gx2[g] - gx1[g]) * (gy2[g] - gy1[g]) for g in range(PAIRS_PER_W)]

        def chunk_body(c, carry):
            bvs, bis = carry
            rows_i = c * LANES + iota
            off = c * LANES
            cx = pred_v[pl.ds(off, LANES)]
            cy = pred_v[pl.ds(N + off, LANES)]
            w = jnp.maximum(pred_v[pl.ds(2 * N + off, LANES)], 0.0001)
            h = jnp.maximum(pred_v[pl.ds(3 * N + off, LANES)], 0.0001)
            x1 = cx - w / 2
            y1 = cy - h / 2
            x2 = cx + w / 2
            y2 = cy + h / 2
            area1 = (x2 - x1) * (y2 - y1)
            new_bvs = []
            new_bis = []
            for g in range(PAIRS_PER_W):
                iw = jnp.maximum(jnp.minimum(x2, gx2[g]) - jnp.maximum(x1, gx1[g]), 0.0)
                ih = jnp.maximum(jnp.minimum(y2, gy2[g]) - jnp.maximum(y1, gy1[g]), 0.0)
                inter = iw * ih
                iou = inter / (area1 + garea[g] - inter)
                upd = iou > bvs[g]
                new_bvs.append(jnp.where(upd, iou, bvs[g]))
                new_bis.append(jnp.where(upd, rows_i, bis[g]))
            return tuple(new_bvs), tuple(new_bis)

        init = (
            tuple(jnp.full((LANES,), -jnp.inf, jnp.float32) for _ in range(PAIRS_PER_W)),
            tuple(jnp.zeros((LANES,), jnp.int32) for _ in range(PAIRS_PER_W)),
        )
        bvs, bis = plsc.parallel_loop(0, CHUNKS, 1, unroll=4, carry=init)(chunk_body)

        valrow = jnp.zeros((LANES,), jnp.float32)
        idxrow = jnp.zeros((LANES,), jnp.int32)
        for g in range(PAIRS_PER_W):
            m = jnp.max(bvs[g])
            mi = jnp.min(jnp.where(bvs[g] == m, bis[g], BIG_I32))
            valrow = jnp.where(iota == g, m, valrow)
            idxrow = jnp.where(iota == g, mi, idxrow)

        valrow_v[...] = valrow
        idxrow_v[...] = idxrow
        pltpu.sync_copy(valrow_v, shval.at[pl.ds(wid * LANES, LANES)])
        pltpu.sync_copy(idxrow_v, shidx.at[pl.ds(wid * LANES, LANES)])
        plsc.subcore_barrier()
        pltpu.sync_copy(shval.at[pl.ds(b * (WPB * LANES), WPB * LANES)], lval_v)
        pltpu.sync_copy(shidx.at[pl.ds(b * (WPB * LANES), WPB * LANES)], lidx_v)

        vrows = [lval_v[pl.ds(r * LANES, LANES)] for r in range(WPB)]
        irows = [lidx_v[pl.ds(r * LANES, LANES)] for r in range(WPB)]
        qrows = [vrows[r] > IOU_THR for r in range(WPB)]

        ni = jnp.int32(0)
        maskv = jnp.zeros((LANES,), jnp.int32)
        for j in range(M):
            r0, g0 = j // PAIRS_PER_W, j % PAIRS_PER_W
            idx_j = irows[r0][g0]
            qual_j = jnp.max(jnp.where((iota == g0) & qrows[r0], 1, 0))
            blk = jnp.zeros((LANES,), jnp.int32)
            for r in range(r0 + 1):
                lane_ok = lane_lt4 if r < r0 else (iota < g0)
                eq = (irows[r] == idx_j) & qrows[r] & lane_ok
                blk = jnp.maximum(blk, jnp.where(eq, 1, 0))
            blocked_j = jnp.max(blk)
            mask_j = qual_j * (1 - blocked_j)
            ni = ni + mask_j
            mine = r0 == jg
            maskv = jnp.where((iota == g0) & mine, mask_j, maskv)
        maskf = maskv.astype(jnp.float32)

        idxv = idxrow
        mcx = plsc.load_gather(pred_v, [idxv])
        mcy = plsc.load_gather(pred_v, [N + idxv])
        mw = plsc.load_gather(pred_v, [2 * N + idxv])
        mh = plsc.load_gather(pred_v, [3 * N + idxv])
        mconf = plsc.load_gather(pred_v, [4 * N + idxv])

        gl = jnp.minimum(iota, PAIRS_PER_W - 1) * 4
        x1g = plsc.load_gather(tgt_v, [gl])
        y1g = plsc.load_gather(tgt_v, [gl + 1])
        x2g = plsc.load_gather(tgt_v, [gl + 2])
        y2g = plsc.load_gather(tgt_v, [gl + 3])

        w = jnp.maximum(mw, 0.0001)
        h = jnp.maximum(mh, 0.0001)
        x1 = mcx - w / 2
        y1 = mcy - h / 2
        x2 = mcx + w / 2
        y2 = mcy + h / 2
        eps = 1e-07
        intsct = (jnp.maximum(jnp.minimum(x2, x2g) - jnp.maximum(x1, x1g), 0.0)
                  * jnp.maximum(jnp.minimum(y2, y2g) - jnp.maximum(y1, y1g), 0.0))
        union = (x2 - x1) * (y2 - y1) + (x2g - x1g) * (y2g - y1g) - intsct + eps
        iou = intsct / union
        dgx = jnp.maximum(x2, x2g) - jnp.minimum(x1, x1g)
        dgy = jnp.maximum(y2, y2g) - jnp.minimum(y1, y1g)
        diag = dgx * dgx + dgy * dgy + eps
        dcx = ((x1g + x2g) - (x1 + x2)) / 2
        dcy = ((y1g + y2g) - (y1 + y2)) / 2
        dist = dcx * dcx + dcy * dcy
        diou = 1.0 - iou + dist / diag
        datan = (_atan((x2g - x1g) / (y2g - y1g))
                 - _atan((x2 - x1) / (y2 - y1)))
        v = jnp.float32(4.0 / (jnp.pi ** 2)) * datan * datan
        alpha = v / (1.0 - iou + v + eps)
        ciou = diou + alpha * v

        nf = jnp.broadcast_to(ni, (LANES,)).astype(jnp.float32)
        box_pre = jnp.where(lane_lt4, ciou * maskf, 0.0)
        box_part = jnp.where(nf > 0.0,
                             box_pre / (jnp.maximum(nf, 1.0) * jnp.float32(B)),
                             0.0)

        l0m, l1m = _focal01(mconf)
        corr_part = maskf * (l1m - l0m) * jnp.float32(1.0 / (N * B))

        w0 = jg * CONF_SLICE

        def conf_body(c, acc):
            x = pred_v[pl.ds(4 * N + w0 + c * LANES, LANES)]
            l0, _ = _focal01(x)
            in_rng = (c * LANES + iota) < CONF_SLICE
            return acc + jnp.where(in_rng, l0, 0.0)

        facc = plsc.parallel_loop(
            0, CONF_CHUNKS, 1, unroll=2,
            carry=jnp.zeros((LANES,), jnp.float32))(conf_body)
        focal_part = facc * jnp.float32(1.0 / (N * B))

        part_v[...] = focal_part + corr_part + box_part
        pltpu.sync_copy(part_v, part_out.at[wid])

    return _sc_loss


def kernel(preds, targets):
    preds_t = jnp.transpose(preds, (2, 0, 1))
    parts = _get_sc_loss()(preds_t.reshape(5 * B * N), targets.reshape(B, M * 4))
    return jnp.sum(parts)
